# baseline (speedup 1.0000x reference)
import functools

import jax
import jax.numpy as jnp
from jax import lax
from jax.experimental import pallas as pl
from jax.experimental.pallas import tpu as pltpu
from jax.experimental.pallas import tpu_sc as plsc

NC = 2
NS = 16
CH = 64


RB = 4
DI = 8
STEP = 8


def _agg_body(n_rows, cpt, x_hbm, src_hbm, dst_hbm, out0_hbm, out1_hbm,
              src_r, dst_r, rows_v, agg_sh, gsems, ssems, issems, idsems):
    h = rows_v.shape[-1]
    c = lax.axis_index("c")
    s = lax.axis_index("s")
    rpt = ((-(-n_rows // NS) + 7) // 8) * 8
    rbase = jnp.minimum(s * rpt, n_rows - rpt)
    col = c * h

    pltpu.sync_copy(x_hbm.at[pl.ds(rbase, rpt), pl.ds(col, h)],
                    agg_sh.at[pl.ds(rbase, rpt)])
    plsc.subcore_barrier()

    def stage_src(j, sl):
        pltpu.async_copy(src_hbm.at[s, j], src_r.at[sl], issems.at[sl])

    def stage_dst(j, sl):
        pltpu.async_copy(dst_hbm.at[s, j], dst_r.at[sl], idsems.at[sl])

    def start_gather(sl, b):
        pltpu.async_copy(x_hbm.at[src_r.at[sl], pl.ds(col, h)], rows_v.at[b],
                         gsems.at[b])

    def wait_gather(sl, b):
        pltpu.make_async_copy(x_hbm.at[src_r.at[sl], pl.ds(col, h)],
                              rows_v.at[b], gsems.at[b]).wait()

    def start_scatter(sl, b):
        pltpu.async_copy(rows_v.at[b], agg_sh.at[dst_r.at[sl]], ssems.at[b],
                         add=True)

    def wait_scatter(sl, b):
        pltpu.make_async_copy(rows_v.at[b], agg_sh.at[dst_r.at[sl]],
                              ssems.at[b]).wait()

    for t in range(DI):
        stage_src(t, t)
    for t in range(DI - 2):
        stage_dst(t, t)
    for t in range(3):
        pltpu.make_async_copy(src_hbm.at[s, t], src_r.at[t], issems.at[t]).wait()
        start_gather(t, t)

    def outer(g, carry):
        for t in range(STEP):
            j = g * STEP + t
            b = t % RB
            pltpu.make_async_copy(dst_hbm.at[s, j], dst_r.at[t],
                                  idsems.at[t]).wait()
            wait_gather(t, b)

            @pl.when(j >= 1)
            def _():
                wait_scatter((t - 1) % DI, (t - 1) % RB)

            start_scatter(t, b)

            @pl.when(j + DI - 2 < cpt)
            def _():
                stage_dst(j + DI - 2, (t + DI - 2) % DI)

            @pl.when(j + DI < cpt)
            def _():
                stage_src(j + DI, t)

            @pl.when(j + 3 < cpt)
            def _():
                pltpu.make_async_copy(src_hbm.at[s, j + 3],
                                      src_r.at[(t + 3) % DI],
                                      issems.at[(t + 3) % DI]).wait()
                start_gather((t + 3) % DI, (t + 3) % RB)
        return carry

    lax.fori_loop(0, cpt // STEP, outer, 0)
    wait_scatter((cpt - 1) % DI, (cpt - 1) % RB)
    plsc.subcore_barrier()

    @pl.when(c == 0)
    def _():
        pltpu.sync_copy(agg_sh.at[pl.ds(rbase, rpt)], out0_hbm.at[pl.ds(rbase, rpt)])

    @pl.when(c == 1)
    def _():
        pltpu.sync_copy(agg_sh.at[pl.ds(rbase, rpt)], out1_hbm.at[pl.ds(rbase, rpt)])


def _sc_aggregate(x, srcs, dsts):
    n_rows, d = x.shape
    h = d // NC
    cpt = srcs.shape[1]
    mesh = plsc.VectorSubcoreMesh(
        core_axis_name="c", subcore_axis_name="s", num_cores=NC, num_subcores=NS)
    kern = pl.kernel(
        functools.partial(_agg_body, n_rows, cpt),
        out_type=(jax.ShapeDtypeStruct((n_rows, h), jnp.float32),
                  jax.ShapeDtypeStruct((n_rows, h), jnp.float32)),
        mesh=mesh,
        scratch_types=[
            pltpu.VMEM((DI, CH), jnp.int32),
            pltpu.VMEM((DI, CH), jnp.int32),
            pltpu.VMEM((RB, CH, h), jnp.float32),
            pltpu.VMEM_SHARED((n_rows + 8, h), jnp.float32),
            pltpu.SemaphoreType.DMA((RB,)),
            pltpu.SemaphoreType.DMA((RB,)),
            pltpu.SemaphoreType.DMA((DI,)),
            pltpu.SemaphoreType.DMA((DI,)),
        ],
    )
    return kern(x, srcs, dsts)


def _mlp_body(h0_ref, h1_ref, w1a_ref, w1b_ref, b1_ref, w2_ref, b2_ref, o_ref):
    h = jnp.dot(h0_ref[...], w1a_ref[...], preferred_element_type=jnp.float32)
    h = h + jnp.dot(h1_ref[...], w1b_ref[...], preferred_element_type=jnp.float32)
    h = jnp.maximum(h + b1_ref[...], 0.0)
    o_ref[...] = jnp.dot(h, w2_ref[...], preferred_element_type=jnp.float32) + b2_ref[...]


def _mlp(h0, h1, w1a, w1b, b1, w2, b2):
    n_rows, h = h0.shape
    d = w2.shape[0]
    rb = 1000
    grid = (n_rows // rb,)
    return pl.pallas_call(
        _mlp_body,
        grid=grid,
        in_specs=[
            pl.BlockSpec((rb, h), lambda i: (i, 0)),
            pl.BlockSpec((rb, h), lambda i: (i, 0)),
            pl.BlockSpec((h, d), lambda i: (0, 0)),
            pl.BlockSpec((h, d), lambda i: (0, 0)),
            pl.BlockSpec((1, d), lambda i: (0, 0)),
            pl.BlockSpec((d, d), lambda i: (0, 0)),
            pl.BlockSpec((1, d), lambda i: (0, 0)),
        ],
        out_specs=pl.BlockSpec((rb, d), lambda i: (i, 0)),
        out_shape=jax.ShapeDtypeStruct((n_rows, d), jnp.float32),
    )(h0, h1, w1a, w1b, b1, w2, b2)


def kernel(x, edge_index, W1, b1, W2, b2):
    n_rows, d = x.shape
    h = d // 2
    e = edge_index.shape[1]
    cpt = ((-(-e // (NS * CH)) + STEP - 1) // STEP) * STEP
    e_pad = NS * cpt * CH

    src = edge_index[0]
    dst = edge_index[1]
    pad = e_pad - e
    src_p = jnp.concatenate([src, jnp.zeros((pad,), jnp.int32)]).reshape(NS, cpt, CH)
    dst_p = jnp.concatenate([dst, jnp.full((pad,), n_rows, jnp.int32)]).reshape(NS, cpt, CH)

    agg0, agg1 = _sc_aggregate(x, src_p, dst_p)
    return _mlp(agg0, agg1, W1[:h], W1[h:], b1.reshape(1, d), W2,
                b2.reshape(1, d))


# speedup vs baseline: 1.0082x; 1.0082x over previous
import functools

import jax
import jax.numpy as jnp
from jax import lax
from jax.experimental import pallas as pl
from jax.experimental.pallas import tpu as pltpu
from jax.experimental.pallas import tpu_sc as plsc

NC = 2
NS = 16
CH = 64


RB = 4
DI = 8
STEP = 8


def _agg_body(n_rows, cpt, x_hbm, src_hbm, dst_hbm, out0_hbm, out1_hbm,
              src_r, dst_r, rows_v, agg_sh, gsems, ssems, issems, idsems):
    h = rows_v.shape[-1]
    c = lax.axis_index("c")
    s = lax.axis_index("s")
    rpt = ((-(-n_rows // NS) + 7) // 8) * 8
    rbase = jnp.minimum(s * rpt, n_rows - rpt)
    col = c * h

    def stage_src(j, sl):
        pltpu.async_copy(src_hbm.at[s, j], src_r.at[sl], issems.at[sl])

    def stage_dst(j, sl):
        pltpu.async_copy(dst_hbm.at[s, j], dst_r.at[sl], idsems.at[sl])

    def start_gather(sl, b):
        pltpu.async_copy(x_hbm.at[src_r.at[sl], pl.ds(col, h)], rows_v.at[b],
                         gsems.at[b])

    def wait_gather(sl, b):
        pltpu.make_async_copy(x_hbm.at[src_r.at[sl], pl.ds(col, h)],
                              rows_v.at[b], gsems.at[b]).wait()

    def start_scatter(sl, b):
        pltpu.async_copy(rows_v.at[b], agg_sh.at[dst_r.at[sl]], ssems.at[b],
                         add=True)

    def wait_scatter(sl, b):
        pltpu.make_async_copy(rows_v.at[b], agg_sh.at[dst_r.at[sl]],
                              ssems.at[b]).wait()

    for t in range(DI):
        stage_src(t, t)
    for t in range(DI - 2):
        stage_dst(t, t)
    pltpu.sync_copy(x_hbm.at[pl.ds(rbase, rpt), pl.ds(col, h)],
                    agg_sh.at[pl.ds(rbase, rpt)])
    plsc.subcore_barrier()
    for t in range(3):
        pltpu.make_async_copy(src_hbm.at[s, t], src_r.at[t], issems.at[t]).wait()
        start_gather(t, t)

    def outer(g, carry):
        for t in range(STEP):
            j = g * STEP + t
            b = t % RB
            pltpu.make_async_copy(dst_hbm.at[s, j], dst_r.at[t],
                                  idsems.at[t]).wait()
            wait_gather(t, b)

            @pl.when(j >= 1)
            def _():
                wait_scatter((t - 1) % DI, (t - 1) % RB)

            start_scatter(t, b)

            @pl.when(j + DI - 2 < cpt)
            def _():
                stage_dst(j + DI - 2, (t + DI - 2) % DI)

            @pl.when(j + DI < cpt)
            def _():
                stage_src(j + DI, t)

            @pl.when(j + 3 < cpt)
            def _():
                pltpu.make_async_copy(src_hbm.at[s, j + 3],
                                      src_r.at[(t + 3) % DI],
                                      issems.at[(t + 3) % DI]).wait()
                start_gather((t + 3) % DI, (t + 3) % RB)
        return carry

    lax.fori_loop(0, cpt // STEP, outer, 0)
    wait_scatter((cpt - 1) % DI, (cpt - 1) % RB)
    plsc.subcore_barrier()

    @pl.when(c == 0)
    def _():
        pltpu.sync_copy(agg_sh.at[pl.ds(rbase, rpt)], out0_hbm.at[pl.ds(rbase, rpt)])

    @pl.when(c == 1)
    def _():
        pltpu.sync_copy(agg_sh.at[pl.ds(rbase, rpt)], out1_hbm.at[pl.ds(rbase, rpt)])


def _sc_aggregate(x, srcs, dsts):
    n_rows, d = x.shape
    h = d // NC
    cpt = srcs.shape[1]
    mesh = plsc.VectorSubcoreMesh(
        core_axis_name="c", subcore_axis_name="s", num_cores=NC, num_subcores=NS)
    kern = pl.kernel(
        functools.partial(_agg_body, n_rows, cpt),
        out_type=(jax.ShapeDtypeStruct((n_rows, h), jnp.float32),
                  jax.ShapeDtypeStruct((n_rows, h), jnp.float32)),
        mesh=mesh,
        scratch_types=[
            pltpu.VMEM((DI, CH), jnp.int32),
            pltpu.VMEM((DI, CH), jnp.int32),
            pltpu.VMEM((RB, CH, h), jnp.float32),
            pltpu.VMEM_SHARED((n_rows + 8, h), jnp.float32),
            pltpu.SemaphoreType.DMA((RB,)),
            pltpu.SemaphoreType.DMA((RB,)),
            pltpu.SemaphoreType.DMA((DI,)),
            pltpu.SemaphoreType.DMA((DI,)),
        ],
    )
    return kern(x, srcs, dsts)


def _mlp_body(h0_ref, h1_ref, w1a_ref, w1b_ref, b1_ref, w2_ref, b2_ref, o_ref):
    h = jnp.dot(h0_ref[...], w1a_ref[...], preferred_element_type=jnp.float32)
    h = h + jnp.dot(h1_ref[...], w1b_ref[...], preferred_element_type=jnp.float32)
    h = jnp.maximum(h + b1_ref[...], 0.0)
    o_ref[...] = jnp.dot(h, w2_ref[...], preferred_element_type=jnp.float32) + b2_ref[...]


def _mlp(h0, h1, w1a, w1b, b1, w2, b2):
    n_rows, h = h0.shape
    d = w2.shape[0]
    rb = 2000
    grid = (n_rows // rb,)
    return pl.pallas_call(
        _mlp_body,
        grid=grid,
        in_specs=[
            pl.BlockSpec((rb, h), lambda i: (i, 0)),
            pl.BlockSpec((rb, h), lambda i: (i, 0)),
            pl.BlockSpec((h, d), lambda i: (0, 0)),
            pl.BlockSpec((h, d), lambda i: (0, 0)),
            pl.BlockSpec((1, d), lambda i: (0, 0)),
            pl.BlockSpec((d, d), lambda i: (0, 0)),
            pl.BlockSpec((1, d), lambda i: (0, 0)),
        ],
        out_specs=pl.BlockSpec((rb, d), lambda i: (i, 0)),
        out_shape=jax.ShapeDtypeStruct((n_rows, d), jnp.float32),
    )(h0, h1, w1a, w1b, b1, w2, b2)


def kernel(x, edge_index, W1, b1, W2, b2):
    n_rows, d = x.shape
    h = d // 2
    e = edge_index.shape[1]
    cpt = ((-(-e // (NS * CH)) + STEP - 1) // STEP) * STEP
    e_pad = NS * cpt * CH

    src = edge_index[0]
    dst = edge_index[1]
    pad = e_pad - e
    src_p = jnp.concatenate([src, jnp.zeros((pad,), jnp.int32)]).reshape(NS, cpt, CH)
    dst_p = jnp.concatenate([dst, jnp.full((pad,), n_rows, jnp.int32)]).reshape(NS, cpt, CH)

    agg0, agg1 = _sc_aggregate(x, src_p, dst_p)
    return _mlp(agg0, agg1, W1[:h], W1[h:], b1.reshape(1, d), W2,
                b2.reshape(1, d))
